# baseline (speedup 1.0000x reference)
import functools

import jax
import jax.numpy as jnp
from jax import lax
from jax.experimental import pallas as pl
from jax.experimental.pallas import tpu as pltpu
from jax.experimental.pallas import tpu_sc as plsc

N_NODES = 100000
MAX_Z = 89
NUM_GRAPHS = 1024

_NC = 2
_NS = 16
_NW = _NC * _NS
_LANES = 16
_CHUNK = 3136
_PAD_N = _NW * _CHUNK
_ACC = NUM_GRAPHS + _LANES
_ACC_STRIDE = _ACC + 1


def _matvec_body(o_ref, a_ref, out_ref):
    out_ref[...] = jnp.dot(o_ref[...], a_ref[...],
                           preferred_element_type=jnp.float32)


def _matvec(node_attr, offset_row):
    nt = node_attr.T
    block = 25088
    grid = _PAD_N // block
    return pl.pallas_call(
        _matvec_body,
        grid=(grid,),
        in_specs=[
            pl.BlockSpec((1, MAX_Z), lambda i: (0, 0)),
            pl.BlockSpec((MAX_Z, block), lambda i: (0, i)),
        ],
        out_specs=pl.BlockSpec((1, block), lambda i: (0, i)),
        out_shape=jax.ShapeDtypeStruct((1, _PAD_N), jnp.float32),
    )(offset_row, nt)


_TAIL = N_NODES - (_NW - 1) * _CHUNK


def _segsum_body(v_hbm, ids_hbm, out_hbm, v_v, ids_v, acc2_v, acc_v):
    wid = lax.axis_index("c") * _NS + lax.axis_index("s")
    base = wid * _CHUNK
    pltpu.sync_copy(v_hbm.at[pl.ds(base, _CHUNK)], v_v)
    @pl.when(wid < _NW - 1)
    def _():
        pltpu.sync_copy(ids_hbm.at[pl.ds(base, _CHUNK)], ids_v)

    @pl.when(wid == _NW - 1)
    def _():
        pltpu.sync_copy(ids_hbm.at[pl.ds(base, _TAIL)], ids_v.at[pl.ds(0, _TAIL)])
        dump = jnp.full((_LANES,), NUM_GRAPHS, jnp.int32)
        for j in range(_TAIL // _LANES, _CHUNK // _LANES):
            ids_v[pl.ds(j * _LANES, _LANES)] = dump

    zeros = jnp.zeros((_LANES,), jnp.float32)
    for r in range(_LANES):
        def _zero(j, carry, r=r):
            acc2_v[r, pl.ds(j * _LANES, _LANES)] = zeros
            return carry

        lax.fori_loop(0, _ACC // _LANES, _zero, 0)

    row = lax.iota(jnp.int32, _LANES)

    def _accum(j, carry):
        idx = ids_v[pl.ds(j * _LANES, _LANES)]
        x = v_v[pl.ds(j * _LANES, _LANES)]
        plsc.addupdate_scatter(acc2_v, [row, idx], x)
        return carry

    lax.fori_loop(0, _CHUNK // _LANES, _accum, 0)

    def _fold(j, carry):
        s = acc2_v[0, pl.ds(j * _LANES, _LANES)]
        for r in range(1, _LANES):
            s = s + acc2_v[r, pl.ds(j * _LANES, _LANES)]
        acc_v[pl.ds(j * _LANES, _LANES)] = s
        return carry

    lax.fori_loop(0, NUM_GRAPHS // _LANES, _fold, 0)
    pltpu.sync_copy(acc_v.at[pl.ds(0, NUM_GRAPHS)], out_hbm.at[wid])


def _segsum(v_pad, ids_pad):
    mesh = plsc.VectorSubcoreMesh(core_axis_name="c", subcore_axis_name="s")
    fn = functools.partial(
        pl.kernel,
        mesh=mesh,
        out_type=jax.ShapeDtypeStruct((_NW, NUM_GRAPHS), jnp.float32),
        scratch_types=[
            pltpu.VMEM((_CHUNK,), jnp.float32),
            pltpu.VMEM((_CHUNK,), jnp.int32),
            pltpu.VMEM((_LANES, _ACC_STRIDE), jnp.float32),
            pltpu.VMEM((NUM_GRAPHS,), jnp.float32),
        ],
        compiler_params=pltpu.CompilerParams(needs_layout_passes=False),
    )(_segsum_body)
    return fn(v_pad, ids_pad)


def _combine_body(p_ref, out_ref):
    out_ref[...] = jnp.sum(p_ref[...], axis=0, keepdims=True)


def _combine(partials):
    return pl.pallas_call(
        _combine_body,
        out_shape=jax.ShapeDtypeStruct((1, NUM_GRAPHS), jnp.float32),
    )(partials)


def kernel(node_attr, segment_ids, property_offset):
    ids = segment_ids.astype(jnp.int32)
    v_pad = _matvec(node_attr, property_offset.reshape(1, MAX_Z)).reshape(-1)
    partials = _segsum(v_pad, ids)
    return _combine(partials).reshape(NUM_GRAPHS)


# speedup vs baseline: 1.1384x; 1.1384x over previous
import functools

import jax
import jax.numpy as jnp
from jax import lax
from jax.experimental import pallas as pl
from jax.experimental.pallas import tpu as pltpu
from jax.experimental.pallas import tpu_sc as plsc

N_NODES = 100000
MAX_Z = 89
NUM_GRAPHS = 1024

_NC = 2
_NS = 16
_NW = _NC * _NS
_LANES = 16
_CHUNK = 3136
_PAD_N = _NW * _CHUNK
_ACC = NUM_GRAPHS + _LANES
_ACC_STRIDE = _ACC + 1


def _matvec_body(o_ref, a_ref, out_ref):
    out_ref[...] = jnp.dot(o_ref[...], a_ref[...],
                           preferred_element_type=jnp.float32)


def _matvec(node_attr, offset_row):
    nt = node_attr.T
    block = 25088
    grid = _PAD_N // block
    return pl.pallas_call(
        _matvec_body,
        grid=(grid,),
        in_specs=[
            pl.BlockSpec((1, MAX_Z), lambda i: (0, 0)),
            pl.BlockSpec((MAX_Z, block), lambda i: (0, i)),
        ],
        out_specs=pl.BlockSpec((1, block), lambda i: (0, i)),
        out_shape=jax.ShapeDtypeStruct((1, _PAD_N), jnp.float32),
    )(offset_row, nt)


_TAIL = N_NODES - (_NW - 1) * _CHUNK


def _segsum_body(v_hbm, ids_hbm, out_hbm, v_v, ids_v, acc_v):
    wid = lax.axis_index("c") * _NS + lax.axis_index("s")
    base = wid * _CHUNK
    pltpu.sync_copy(v_hbm.at[pl.ds(base, _CHUNK)], v_v)
    @pl.when(wid < _NW - 1)
    def _():
        pltpu.sync_copy(ids_hbm.at[pl.ds(base, _CHUNK)], ids_v)

    @pl.when(wid == _NW - 1)
    def _():
        pltpu.sync_copy(ids_hbm.at[pl.ds(base, _TAIL)], ids_v.at[pl.ds(0, _TAIL)])
        dump = jnp.full((_LANES,), NUM_GRAPHS, jnp.int32)
        for j in range(_TAIL // _LANES, _CHUNK // _LANES):
            ids_v[pl.ds(j * _LANES, _LANES)] = dump

    zeros = jnp.zeros((_LANES,), jnp.float32)

    @plsc.parallel_loop(0, _ACC // _LANES, unroll=4)
    def _zero(j):
        acc_v[pl.ds(j * _LANES, _LANES)] = zeros

    @plsc.parallel_loop(0, _CHUNK // _LANES, unroll=4)
    def _accum(j):
        idx = ids_v[pl.ds(j * _LANES, _LANES)]
        x = v_v[pl.ds(j * _LANES, _LANES)]
        plsc.addupdate_scatter(acc_v, [idx], x)

    pltpu.sync_copy(acc_v.at[pl.ds(0, NUM_GRAPHS)], out_hbm.at[wid])


def _segsum(v_pad, ids_pad):
    mesh = plsc.VectorSubcoreMesh(core_axis_name="c", subcore_axis_name="s")
    fn = functools.partial(
        pl.kernel,
        mesh=mesh,
        out_type=jax.ShapeDtypeStruct((_NW, NUM_GRAPHS), jnp.float32),
        scratch_types=[
            pltpu.VMEM((_CHUNK,), jnp.float32),
            pltpu.VMEM((_CHUNK,), jnp.int32),
            pltpu.VMEM((_ACC,), jnp.float32),
        ],
        compiler_params=pltpu.CompilerParams(needs_layout_passes=False),
    )(_segsum_body)
    return fn(v_pad, ids_pad)


def _combine_body(p_ref, out_ref):
    out_ref[...] = jnp.sum(p_ref[...], axis=0, keepdims=True)


def _combine(partials):
    return pl.pallas_call(
        _combine_body,
        out_shape=jax.ShapeDtypeStruct((1, NUM_GRAPHS), jnp.float32),
    )(partials)


def kernel(node_attr, segment_ids, property_offset):
    ids = segment_ids.astype(jnp.int32)
    v_pad = _matvec(node_attr, property_offset.reshape(1, MAX_Z)).reshape(-1)
    partials = _segsum(v_pad, ids)
    return _combine(partials).reshape(NUM_GRAPHS)
